# baseline (speedup 1.0000x reference)
import functools

import jax
import jax.numpy as jnp
from jax import lax
from jax.experimental import pallas as pl
from jax.experimental.pallas import tpu as pltpu
from jax.experimental.pallas import tpu_sc as plsc

_NC = 2
_NS = 16
_CH = 96


def _pad_rows(n):
  return ((n + 1) + 127) // 128 * 128


def _edge_iters(e):
  nw = _NC * _NS
  return ((e + nw * _CH - 1) // (nw * _CH) + 3) // 4 * 4


def _sc_degree(dst3, cst, npad, iters):
  rpt = npad // _NS

  @functools.partial(
      pl.kernel,
      out_type=jax.ShapeDtypeStruct((_NC, npad, 128), jnp.float32),
      mesh=plsc.VectorSubcoreMesh(core_axis_name="c", subcore_axis_name="s"),
      scratch_types=[
          pltpu.VMEM((iters, _CH), jnp.int32),
          pltpu.VMEM((_CH, 128), jnp.float32),
          pltpu.SemaphoreType.DMA,
          pltpu.VMEM_SHARED((npad, 128), jnp.float32),
      ],
  )
  def k(dst_hbm, cst_hbm, out_hbm, idx_v, ones_v, sem, acc_sh):
    cid = lax.axis_index("c")
    sid = lax.axis_index("s")
    wid = sid * _NC + cid

    pltpu.sync_copy(dst_hbm.at[wid], idx_v)
    pltpu.sync_copy(cst_hbm.at[pl.ds(sid * rpt, rpt)],
                    acc_sh.at[pl.ds(sid * rpt, rpt)])
    pltpu.sync_copy(cst_hbm.at[pl.ds(npad, _CH)], ones_v)
    plsc.subcore_barrier()

    def fire(i, _):
      pltpu.async_copy(ones_v, acc_sh.at[idx_v.at[i]], sem, add=True)
      return 0

    lax.fori_loop(0, iters, fire, 0)

    def drain(i, _):
      pltpu.make_async_copy(ones_v, acc_sh.at[idx_v.at[0]], sem).wait()
      return 0

    lax.fori_loop(0, iters, drain, 0)
    plsc.subcore_barrier()
    pltpu.sync_copy(
        acc_sh.at[pl.ds(sid * rpt, rpt)],
        out_hbm.at[cid, pl.ds(sid * rpt, rpt)],
    )

  return k(dst3, cst)


def _sc_edge_sum(vals, esd, zeros, npad, iters):
  rpt = npad // _NS

  @functools.partial(
      pl.kernel,
      out_type=jax.ShapeDtypeStruct((_NC, npad, 128), jnp.float32),
      mesh=plsc.VectorSubcoreMesh(core_axis_name="c", subcore_axis_name="s"),
      scratch_types=[
          pltpu.VMEM((2, _CH), jnp.int32),
          pltpu.VMEM((2, _CH), jnp.int32),
          pltpu.VMEM((2, _CH), jnp.int32),
          pltpu.VMEM((2, _CH), jnp.int32),
          pltpu.VMEM((_CH, 128), jnp.float32),
          pltpu.VMEM((_CH, 128), jnp.float32),
          pltpu.SemaphoreType.DMA,
          pltpu.SemaphoreType.DMA,
          pltpu.SemaphoreType.DMA,
          pltpu.SemaphoreType.DMA,
          pltpu.SemaphoreType.DMA,
          pltpu.SemaphoreType.DMA,
          pltpu.SemaphoreType.DMA,
          pltpu.SemaphoreType.DMA,
          pltpu.VMEM_SHARED((npad, 128), jnp.float32),
      ],
  )
  def k(vals_hbm, esd_hbm, zeros_hbm, out_hbm, i0, i1, i2, i3,
        r0, r1, q0, q1, q2, q3, g0, g1, s0, s1, acc_sh):
    cid = lax.axis_index("c")
    sid = lax.axis_index("s")
    wid = sid * _NC + cid
    ibuf = (i0, i1, i2, i3)
    isem = (q0, q1, q2, q3)
    rows = (r0, r1)
    gsem = (g0, g1)
    ssem = (s0, s1)

    pltpu.sync_copy(zeros_hbm.at[pl.ds(sid * rpt, rpt)],
                    acc_sh.at[pl.ds(sid * rpt, rpt)])

    def idx_load(j, b):
      pltpu.async_copy(esd_hbm.at[wid, j], ibuf[b], isem[b])

    def idx_wait(j, b):
      pltpu.make_async_copy(esd_hbm.at[wid, j], ibuf[b], isem[b]).wait()

    def gather(j, b, ib):
      pltpu.async_copy(vals_hbm.at[ibuf[ib].at[0]], rows[b], gsem[b])

    def gather_wait(j, b, ib):
      pltpu.make_async_copy(vals_hbm.at[ibuf[ib].at[0]], rows[b],
                            gsem[b]).wait()

    def scatter(j, b, ib):
      pltpu.async_copy(rows[b], acc_sh.at[ibuf[ib].at[1]], ssem[b],
                       add=True)

    def scatter_wait(j, b, ib):
      pltpu.make_async_copy(rows[b], acc_sh.at[ibuf[ib].at[1]],
                            ssem[b]).wait()

    idx_load(0, 0)
    idx_load(1, 1)
    idx_load(2, 2)
    idx_wait(0, 0)
    gather(0, 0, 0)
    plsc.subcore_barrier()

    def group(kk, _):
      for u in range(4):
        j = 4 * kk + u
        u2 = u % 2
        o2 = (u + 1) % 2
        n4 = (u + 3) % 4
        g4 = (u + 1) % 4

        @pl.when(j >= 1)
        def _():
          scatter_wait(j - 1, o2, n4)

        @pl.when(j + 3 < iters)
        def _():
          idx_load(j + 3, n4)

        @pl.when(j + 1 < iters)
        def _():
          idx_wait(j + 1, g4)
          gather(j + 1, o2, g4)

        gather_wait(j, u2, u)
        scatter(j, u2, u)
      return 0

    lax.fori_loop(0, iters // 4, group, 0)
    scatter_wait(iters - 1, (iters - 1) % 2, (iters - 1) % 4)
    plsc.subcore_barrier()
    pltpu.sync_copy(
        acc_sh.at[pl.ds(sid * rpt, rpt)],
        out_hbm.at[cid, pl.ds(sid * rpt, rpt)],
    )

  return k(vals, esd, zeros)


def _tc_bn0_body(x_ref, g_ref, b_ref, h_ref):
  xv = x_ref[...]
  mean = jnp.mean(xv, axis=0, keepdims=True)
  var = jnp.mean((xv - mean) ** 2, axis=0, keepdims=True)
  h_ref[...] = (xv - mean) * lax.rsqrt(var + 1e-5) * g_ref[...] + b_ref[...]


def _tc_scale_body(h_ref, d_ref, m_ref, dv_ref):
  n = h_ref.shape[0]
  npad = m_ref.shape[0]
  deg = d_ref[0, 0:n, 0:1] + d_ref[1, 0:n, 0:1] + 1.0
  dinv = lax.rsqrt(deg)
  m = h_ref[...] * dinv
  m_ref[...] = jnp.concatenate(
      [m, jnp.zeros((npad - n, m.shape[1]), m.dtype)], axis=0)
  dv_ref[...] = jnp.broadcast_to(dinv, dv_ref.shape)


def _tc_mid_body(p_ref, m_ref, dv_ref, w1_ref, b1_ref, g1_ref, be1_ref,
                 wr_ref, wn_ref, bg_ref, z_ref, r_ref):
  n = dv_ref.shape[0]
  npad = m_ref.shape[0]
  pre = (p_ref[0, 0:n] + p_ref[1, 0:n] + m_ref[0:n]) * dv_ref[:, 0:1]
  h1 = jnp.dot(pre, w1_ref[...], preferred_element_type=jnp.float32)
  a = jnp.maximum(h1 + b1_ref[...], 0.0)
  mean = jnp.mean(a, axis=0, keepdims=True)
  var = jnp.mean((a - mean) ** 2, axis=0, keepdims=True)
  h2 = (a - mean) * lax.rsqrt(var + 1e-5) * g1_ref[...] + be1_ref[...]
  zz = jnp.dot(h2, wn_ref[...], preferred_element_type=jnp.float32)
  zw = jnp.concatenate([zz, jnp.zeros_like(zz)], axis=1)
  z_ref[...] = jnp.concatenate(
      [zw, jnp.zeros((npad - n, zw.shape[1]), zw.dtype)], axis=0)
  r_ref[...] = (
      jnp.dot(h2, wr_ref[...], preferred_element_type=jnp.float32)
      + bg_ref[...]
  )


def _tc_head_body(r_ref, q_ref, w1_ref, b1_ref, w2_ref, b2_ref, o_ref):
  n, f2 = r_ref.shape
  out = jnp.maximum(
      r_ref[...] + q_ref[0, 0:n, 0:f2] + q_ref[1, 0:n, 0:f2], 0.0)
  h = jnp.maximum(
      jnp.dot(out, w1_ref[...], preferred_element_type=jnp.float32)
      + b1_ref[...], 0.0)
  o_ref[...] = (
      jnp.dot(h, w2_ref[...], preferred_element_type=jnp.float32)
      + b2_ref[...]
  )


def kernel(x, edge_index, bn0_gamma, bn0_beta, gcn1_W, gcn1_b, bn1_gamma,
           bn1_beta, gc2_W_root, gc2_W_nbr, gc2_b, fc1_W, fc1_b, fc2_W,
           fc2_b):
  n, c_in = x.shape
  e = edge_index.shape[1]
  f2 = gc2_W_root.shape[1]
  nc = fc2_W.shape[1]
  npad = _pad_rows(n)
  nw = _NC * _NS
  iters = _edge_iters(e)
  ep = nw * iters * _CH
  pad_edge = jnp.full((1, ep - e), n, jnp.int32)
  eip = jnp.concatenate([edge_index, jnp.broadcast_to(pad_edge, (2, ep - e))],
                        axis=1)
  src3 = eip[0].reshape(nw, iters, _CH)
  dst3 = eip[1].reshape(nw, iters, _CH)
  esd = jnp.stack([src3, dst3], axis=2)

  zeros = jnp.zeros((npad, 128), jnp.float32)
  cst = jnp.concatenate([zeros, jnp.ones((_CH, 128), jnp.float32)], axis=0)

  dcnt = _sc_degree(dst3, cst, npad, iters)

  h0 = pl.pallas_call(
      _tc_bn0_body,
      out_shape=jax.ShapeDtypeStruct((n, c_in), jnp.float32),
  )(x, bn0_gamma.reshape(1, -1), bn0_beta.reshape(1, -1))

  m, dv = pl.pallas_call(
      _tc_scale_body,
      out_shape=(
          jax.ShapeDtypeStruct((npad, c_in), jnp.float32),
          jax.ShapeDtypeStruct((n, 8), jnp.float32),
      ),
  )(h0, dcnt)

  p = _sc_edge_sum(m, esd, zeros, npad, iters)

  z, r = pl.pallas_call(
      _tc_mid_body,
      out_shape=(
          jax.ShapeDtypeStruct((npad, 2 * f2), jnp.float32),
          jax.ShapeDtypeStruct((n, f2), jnp.float32),
      ),
  )(p, m, dv, gcn1_W, gcn1_b.reshape(1, -1), bn1_gamma.reshape(1, -1),
    bn1_beta.reshape(1, -1), gc2_W_root, gc2_W_nbr, gc2_b.reshape(1, -1))

  q = _sc_edge_sum(z, esd, zeros, npad, iters)

  logits = pl.pallas_call(
      _tc_head_body,
      out_shape=jax.ShapeDtypeStruct((n, nc), jnp.float32),
  )(r, q, fc1_W, fc1_b.reshape(1, -1), fc2_W, fc2_b.reshape(1, -1))

  return logits
